# baseline (speedup 1.0000x reference)
import numpy as np
import jax
import jax.numpy as jnp
from jax.experimental import pallas as pl
from jax.experimental.pallas import tpu as pltpu

_B, _H, _S, _DH, _BLK = 2, 16, 4096, 64, 64
_NB = _S // _BLK
_R = 3
_M = _NB - 2
_VP = 128
_SCALE = 1.0 / np.sqrt(_DH)


def _dot_t(a, b):
    return jax.lax.dot_general(
        a, b, (((1,), (1,)), ((), ())), preferred_element_type=jnp.float32
    )


def _bigbird_kernel(rand_ref, q_ref, k_ref, vp_ref, o_ref,
                    kr_ref, kg_ref, vg_ref):
    h = pl.program_id(0)

    for b in range(_B):
        kg_ref[b, 0:_BLK] = k_ref[b, 0, 0:_BLK]
        kg_ref[b, _BLK:2 * _BLK] = k_ref[b, 0, _S - _BLK:_S]
        vg_ref[b, 0:_BLK] = vp_ref[b, 0, 0:_BLK]
        vg_ref[b, _BLK:2 * _BLK] = vp_ref[b, 0, _S - _BLK:_S]

    for b in range(_B):
        qg = jnp.concatenate(
            [q_ref[b, 0, 0:_BLK], q_ref[b, 0, _S - _BLK:_S]], axis=0
        )
        s = _dot_t(qg, k_ref[b, 0])
        s = s - jnp.max(s, axis=-1, keepdims=True)
        p = jnp.exp(s).astype(jnp.bfloat16)
        res = jnp.dot(p, vp_ref[b, 0], preferred_element_type=jnp.float32)
        og = res[:, 0:_DH] / res[:, _DH:_DH + 1]
        o_ref[b, 0, 0:_BLK] = og[0:_BLK]
        o_ref[b, 0, _S - _BLK:_S] = og[_BLK:]

    def one_row(m):
        r0 = rand_ref[h, m, 0]
        r1 = rand_ref[h, m, 1]
        r2 = rand_ref[h, m, 2]
        for b in range(_B):
            kr_ref[b, 0:_BLK] = k_ref[b, 0, pl.ds(r0 * _BLK, _BLK)]
            kr_ref[b, _BLK:2 * _BLK] = k_ref[b, 0, pl.ds(r1 * _BLK, _BLK)]
            kr_ref[b, 2 * _BLK:3 * _BLK] = k_ref[b, 0, pl.ds(r2 * _BLK, _BLK)]
        for b in range(_B):
            qm = q_ref[b, 0, pl.ds((m + 1) * _BLK, _BLK)]
            s_g = _dot_t(qm, kg_ref[b])
            s_w = _dot_t(qm, k_ref[b, 0, pl.ds(m * _BLK, 3 * _BLK)])
            s_r = _dot_t(qm, kr_ref[b])
            mx = jnp.maximum(
                jnp.max(s_g, axis=-1, keepdims=True),
                jnp.maximum(
                    jnp.max(s_w, axis=-1, keepdims=True),
                    jnp.max(s_r, axis=-1, keepdims=True),
                ),
            )
            e_g = jnp.exp(s_g - mx).astype(jnp.bfloat16)
            e_w = jnp.exp(s_w - mx).astype(jnp.bfloat16)
            e_r = jnp.exp(s_r - mx).astype(jnp.bfloat16)
            acc = jnp.dot(e_g, vg_ref[b], preferred_element_type=jnp.float32)
            acc = acc + jnp.dot(
                e_w, vp_ref[b, 0, pl.ds(m * _BLK, 3 * _BLK)],
                preferred_element_type=jnp.float32)
            acc = acc + jnp.dot(
                e_r[:, 0:_BLK], vp_ref[b, 0, pl.ds(r0 * _BLK, _BLK)],
                preferred_element_type=jnp.float32)
            acc = acc + jnp.dot(
                e_r[:, _BLK:2 * _BLK], vp_ref[b, 0, pl.ds(r1 * _BLK, _BLK)],
                preferred_element_type=jnp.float32)
            acc = acc + jnp.dot(
                e_r[:, 2 * _BLK:3 * _BLK], vp_ref[b, 0, pl.ds(r2 * _BLK, _BLK)],
                preferred_element_type=jnp.float32)
            o_ref[b, 0, pl.ds((m + 1) * _BLK, _BLK)] = (
                acc[:, 0:_DH] / acc[:, _DH:_DH + 1]
            )

    def body(i, carry):
        one_row(2 * i)
        one_row(2 * i + 1)
        return carry

    jax.lax.fori_loop(0, _M // 2, body, 0)


def kernel(q, k, v, rand_attn):
    rand = rand_attn.astype(jnp.int32)
    qs = (q * _SCALE).astype(jnp.bfloat16)
    kb = k.astype(jnp.bfloat16)
    vp = jnp.concatenate(
        [
            v.astype(jnp.bfloat16),
            jnp.ones((_B, _H, _S, 1), jnp.bfloat16),
            jnp.zeros((_B, _H, _S, _VP - _DH - 1), jnp.bfloat16),
        ],
        axis=-1,
    )

    def _spec(h, rand_ref):
        return (0, h, 0, 0)

    out = pl.pallas_call(
        _bigbird_kernel,
        grid_spec=pltpu.PrefetchScalarGridSpec(
            num_scalar_prefetch=1,
            grid=(_H,),
            in_specs=[
                pl.BlockSpec((_B, 1, _S, _DH), _spec),
                pl.BlockSpec((_B, 1, _S, _DH), _spec),
                pl.BlockSpec((_B, 1, _S, _VP), _spec),
            ],
            out_specs=pl.BlockSpec((_B, 1, _S, _DH), _spec),
            scratch_shapes=[
                pltpu.VMEM((_B, 3 * _BLK, _DH), jnp.bfloat16),
                pltpu.VMEM((_B, 2 * _BLK, _DH), jnp.bfloat16),
                pltpu.VMEM((_B, 2 * _BLK, _VP), jnp.bfloat16),
            ],
        ),
        out_shape=jax.ShapeDtypeStruct((_B, _H, _S, _DH), jnp.float32),
        compiler_params=pltpu.CompilerParams(
            dimension_semantics=("parallel",),
        ),
    )(rand, qs, kb, vp)
    return out


# speedup vs baseline: 1.1833x; 1.1833x over previous
import numpy as np
import jax
import jax.numpy as jnp
from jax.experimental import pallas as pl
from jax.experimental.pallas import tpu as pltpu

_B, _H, _S, _DH, _BLK = 2, 16, 4096, 64, 64
_NB = _S // _BLK
_R = 3
_M = _NB - 2
_VP = 128
_SCALE = 1.0 / np.sqrt(_DH)


def _dot_t(a, b):
    return jax.lax.dot_general(
        a, b, (((1,), (1,)), ((), ())), preferred_element_type=jnp.float32
    )


def _dot(a, b):
    return jnp.dot(a, b, preferred_element_type=jnp.float32)


def _bigbird_kernel(rand_ref, q_ref, k_ref, v_ref, o_ref,
                    qb_ref, kb_ref, vp_ref, kg_ref, vg_ref):
    h = pl.program_id(1)

    qb_ref[...] = (q_ref[0, 0] * _SCALE).astype(jnp.bfloat16)
    kb_ref[...] = k_ref[0, 0].astype(jnp.bfloat16)
    vp_ref[:, 0:_DH] = v_ref[0, 0].astype(jnp.bfloat16)
    vp_ref[:, _DH:_VP] = jnp.zeros((_S, _VP - _DH), jnp.bfloat16)
    vp_ref[:, _DH:_DH + 1] = jnp.ones((_S, 1), jnp.bfloat16)
    kg_ref[0:_BLK] = kb_ref[0:_BLK]
    kg_ref[_BLK:2 * _BLK] = kb_ref[_S - _BLK:_S]
    vg_ref[0:_BLK] = vp_ref[0:_BLK]
    vg_ref[_BLK:2 * _BLK] = vp_ref[_S - _BLK:_S]

    qg = jnp.concatenate(
        [qb_ref[0:_BLK], qb_ref[_S - _BLK:_S]], axis=0
    )
    s = _dot_t(qg, kb_ref[...])
    s = s - jnp.max(s, axis=-1, keepdims=True)
    p = jnp.exp(s).astype(jnp.bfloat16)
    res = _dot(p, vp_ref[...])
    og = res[:, 0:_DH] / res[:, _DH:_DH + 1]
    o_ref[0, 0, 0:_BLK] = og[0:_BLK]
    o_ref[0, 0, _S - _BLK:_S] = og[_BLK:]

    def one_row(m):
        r0 = rand_ref[h, m, 0]
        r1 = rand_ref[h, m, 1]
        r2 = rand_ref[h, m, 2]
        qm = qb_ref[pl.ds((m + 1) * _BLK, _BLK)]
        s_g = _dot_t(qm, kg_ref[...])
        s_w = _dot_t(qm, kb_ref[pl.ds(m * _BLK, 3 * _BLK)])
        s_0 = _dot_t(qm, kb_ref[pl.ds(r0 * _BLK, _BLK)])
        s_1 = _dot_t(qm, kb_ref[pl.ds(r1 * _BLK, _BLK)])
        s_2 = _dot_t(qm, kb_ref[pl.ds(r2 * _BLK, _BLK)])
        mx = jnp.maximum(
            jnp.maximum(
                jnp.max(s_g, axis=-1, keepdims=True),
                jnp.max(s_w, axis=-1, keepdims=True),
            ),
            jnp.maximum(
                jnp.max(s_0, axis=-1, keepdims=True),
                jnp.maximum(
                    jnp.max(s_1, axis=-1, keepdims=True),
                    jnp.max(s_2, axis=-1, keepdims=True),
                ),
            ),
        )
        acc = _dot(jnp.exp(s_g - mx).astype(jnp.bfloat16), vg_ref[...])
        acc = acc + _dot(jnp.exp(s_w - mx).astype(jnp.bfloat16),
                         vp_ref[pl.ds(m * _BLK, 3 * _BLK)])
        acc = acc + _dot(jnp.exp(s_0 - mx).astype(jnp.bfloat16),
                         vp_ref[pl.ds(r0 * _BLK, _BLK)])
        acc = acc + _dot(jnp.exp(s_1 - mx).astype(jnp.bfloat16),
                         vp_ref[pl.ds(r1 * _BLK, _BLK)])
        acc = acc + _dot(jnp.exp(s_2 - mx).astype(jnp.bfloat16),
                         vp_ref[pl.ds(r2 * _BLK, _BLK)])
        o_ref[0, 0, pl.ds((m + 1) * _BLK, _BLK)] = (
            acc[:, 0:_DH] / acc[:, _DH:_DH + 1]
        )

    def body(i, carry):
        for j in range(8):
            one_row(8 * i + j)
        return carry

    jax.lax.fori_loop(0, 7, body, 0)
    for m in range(56, _M):
        one_row(m)


def kernel(q, k, v, rand_attn):
    rand = rand_attn.astype(jnp.int32)

    def _spec(b, h, rand_ref):
        return (b, h, 0, 0)

    qkv_spec = pl.BlockSpec((1, 1, _S, _DH), _spec)
    out = pl.pallas_call(
        _bigbird_kernel,
        grid_spec=pltpu.PrefetchScalarGridSpec(
            num_scalar_prefetch=1,
            grid=(_B, _H),
            in_specs=[qkv_spec, qkv_spec, qkv_spec],
            out_specs=qkv_spec,
            scratch_shapes=[
                pltpu.VMEM((_S, _DH), jnp.bfloat16),
                pltpu.VMEM((_S, _DH), jnp.bfloat16),
                pltpu.VMEM((_S, _VP), jnp.bfloat16),
                pltpu.VMEM((2 * _BLK, _DH), jnp.bfloat16),
                pltpu.VMEM((2 * _BLK, _VP), jnp.bfloat16),
            ],
        ),
        out_shape=jax.ShapeDtypeStruct((_B, _H, _S, _DH), jnp.float32),
        compiler_params=pltpu.CompilerParams(
            dimension_semantics=("parallel", "parallel"),
        ),
    )(rand, q, k, v)
    return out


# speedup vs baseline: 1.5022x; 1.2696x over previous
import numpy as np
import jax
import jax.numpy as jnp
from jax.experimental import pallas as pl
from jax.experimental.pallas import tpu as pltpu

_B, _H, _S, _DH, _BLK = 2, 16, 4096, 64, 64
_NB = _S // _BLK
_R = 3
_M = _NB - 2
_VP = 128
_SCALE = 1.0 / np.sqrt(_DH)
_BOUND_PAD = 1.01


def _dot_t(a, b):
    return jax.lax.dot_general(
        a, b, (((1,), (1,)), ((), ())), preferred_element_type=jnp.float32
    )


def _dot(a, b):
    return jnp.dot(a, b, preferred_element_type=jnp.float32)


def _bigbird_kernel(rand_ref, q_ref, k_ref, v_ref, o_ref,
                    qb_ref, kb_ref, vp_ref, kg_ref, vg_ref):
    h = pl.program_id(1)

    qb_ref[...] = (q_ref[0, 0] * _SCALE).astype(jnp.bfloat16)
    kb_ref[...] = k_ref[0, 0].astype(jnp.bfloat16)
    vp_ref[:, 0:_DH] = v_ref[0, 0].astype(jnp.bfloat16)
    vp_ref[:, _DH:_VP] = jnp.zeros((_S, _VP - _DH), jnp.bfloat16)
    vp_ref[:, _DH:_DH + 1] = jnp.ones((_S, 1), jnp.bfloat16)
    kg_ref[0:_BLK] = kb_ref[0:_BLK]
    kg_ref[_BLK:2 * _BLK] = kb_ref[_S - _BLK:_S]
    vg_ref[0:_BLK] = vp_ref[0:_BLK]
    vg_ref[_BLK:2 * _BLK] = vp_ref[_S - _BLK:_S]

    kf = k_ref[0, 0]
    kmax = jnp.sqrt(jnp.max(jnp.sum(kf * kf, axis=-1))) * _BOUND_PAD

    def row_bound(q_rows):
        qf = q_rows.astype(jnp.float32)
        return jnp.sqrt(
            jnp.sum(qf * qf, axis=-1, keepdims=True)
        ) * (kmax * _BOUND_PAD)

    qg = jnp.concatenate(
        [qb_ref[0:_BLK], qb_ref[_S - _BLK:_S]], axis=0
    )
    mx_g = row_bound(qg)
    s = _dot_t(qg, kb_ref[...])
    p = jnp.exp(s - mx_g).astype(jnp.bfloat16)
    res = _dot(p, vp_ref[...])
    og = res[:, 0:_DH] / res[:, _DH:_DH + 1]
    o_ref[0, 0, 0:_BLK] = og[0:_BLK]
    o_ref[0, 0, _S - _BLK:_S] = og[_BLK:]

    def one_row(m):
        r0 = rand_ref[h, m, 0]
        r1 = rand_ref[h, m, 1]
        r2 = rand_ref[h, m, 2]
        qm = qb_ref[pl.ds((m + 1) * _BLK, _BLK)]
        mx = row_bound(qm)
        s_g = _dot_t(qm, kg_ref[...])
        s_w = _dot_t(qm, kb_ref[pl.ds(m * _BLK, 3 * _BLK)])
        s_0 = _dot_t(qm, kb_ref[pl.ds(r0 * _BLK, _BLK)])
        s_1 = _dot_t(qm, kb_ref[pl.ds(r1 * _BLK, _BLK)])
        s_2 = _dot_t(qm, kb_ref[pl.ds(r2 * _BLK, _BLK)])
        acc = _dot(jnp.exp(s_g - mx).astype(jnp.bfloat16), vg_ref[...])
        acc = acc + _dot(jnp.exp(s_w - mx).astype(jnp.bfloat16),
                         vp_ref[pl.ds(m * _BLK, 3 * _BLK)])
        acc = acc + _dot(jnp.exp(s_0 - mx).astype(jnp.bfloat16),
                         vp_ref[pl.ds(r0 * _BLK, _BLK)])
        acc = acc + _dot(jnp.exp(s_1 - mx).astype(jnp.bfloat16),
                         vp_ref[pl.ds(r1 * _BLK, _BLK)])
        acc = acc + _dot(jnp.exp(s_2 - mx).astype(jnp.bfloat16),
                         vp_ref[pl.ds(r2 * _BLK, _BLK)])
        o_ref[0, 0, pl.ds((m + 1) * _BLK, _BLK)] = (
            acc[:, 0:_DH] / acc[:, _DH:_DH + 1]
        )

    def body(i, carry):
        for j in range(8):
            one_row(8 * i + j)
        return carry

    jax.lax.fori_loop(0, 7, body, 0)
    for m in range(56, _M):
        one_row(m)


def kernel(q, k, v, rand_attn):
    rand = rand_attn.astype(jnp.int32)

    def _spec(b, h, rand_ref):
        return (b, h, 0, 0)

    qkv_spec = pl.BlockSpec((1, 1, _S, _DH), _spec)
    out = pl.pallas_call(
        _bigbird_kernel,
        grid_spec=pltpu.PrefetchScalarGridSpec(
            num_scalar_prefetch=1,
            grid=(_B, _H),
            in_specs=[qkv_spec, qkv_spec, qkv_spec],
            out_specs=qkv_spec,
            scratch_shapes=[
                pltpu.VMEM((_S, _DH), jnp.bfloat16),
                pltpu.VMEM((_S, _DH), jnp.bfloat16),
                pltpu.VMEM((_S, _VP), jnp.bfloat16),
                pltpu.VMEM((2 * _BLK, _DH), jnp.bfloat16),
                pltpu.VMEM((2 * _BLK, _VP), jnp.bfloat16),
            ],
        ),
        out_shape=jax.ShapeDtypeStruct((_B, _H, _S, _DH), jnp.float32),
        compiler_params=pltpu.CompilerParams(
            dimension_semantics=("parallel", "parallel"),
        ),
    )(rand, q, k, v)
    return out


# speedup vs baseline: 1.7746x; 1.1813x over previous
import numpy as np
import jax
import jax.numpy as jnp
from jax.experimental import pallas as pl
from jax.experimental.pallas import tpu as pltpu

_B, _H, _S, _DH, _BLK = 2, 16, 4096, 64, 64
_NB = _S // _BLK
_R = 3
_M = _NB - 2
_VP = 128
_SCALE = 1.0 / np.sqrt(_DH)
_BOUND_PAD = 1.01


def _dot_t(a, b):
    return jax.lax.dot_general(
        a, b, (((1,), (1,)), ((), ())), preferred_element_type=jnp.float32
    )


def _dot(a, b):
    return jnp.dot(a, b, preferred_element_type=jnp.float32)


def _bigbird_kernel(rand_ref, q_ref, k_ref, v_ref, o_ref,
                    kb_ref, vp_ref, kg_ref, vg_ref):
    h = pl.program_id(1)

    kb_ref[...] = k_ref[0, 0].astype(jnp.bfloat16)
    vp_ref[:, 0:_DH] = v_ref[0, 0].astype(jnp.bfloat16)
    vp_ref[:, _DH:_VP] = jnp.zeros((_S, _VP - _DH), jnp.bfloat16)
    vp_ref[:, _DH:_DH + 1] = jnp.ones((_S, 1), jnp.bfloat16)
    kg_ref[0:_BLK] = kb_ref[0:_BLK]
    kg_ref[_BLK:2 * _BLK] = kb_ref[_S - _BLK:_S]
    vg_ref[0:_BLK] = vp_ref[0:_BLK]
    vg_ref[_BLK:2 * _BLK] = vp_ref[_S - _BLK:_S]

    kf = k_ref[0, 0]
    kmax = jnp.sqrt(jnp.max(jnp.sum(kf * kf, axis=-1))) * _BOUND_PAD

    def row_bound(q_rows):
        qf = q_rows.astype(jnp.float32)
        return jnp.sqrt(
            jnp.sum(qf * qf, axis=-1, keepdims=True)
        ) * (kmax * _BOUND_PAD)

    qg = jnp.concatenate(
        [q_ref[0, 0, 0:_BLK], q_ref[0, 0, _S - _BLK:_S]], axis=0
    )
    qg = (qg * _SCALE).astype(jnp.bfloat16)
    mx_g = row_bound(qg)
    s = _dot_t(qg, kb_ref[...])
    p = jnp.exp(s - mx_g).astype(jnp.bfloat16)
    res = _dot(p, vp_ref[...])
    og = res[:, 0:_DH] / res[:, _DH:_DH + 1]
    o_ref[0, 0, 0:_BLK] = og[0:_BLK]
    o_ref[0, 0, _S - _BLK:_S] = og[_BLK:]

    def qk_phase(m):
        r0 = rand_ref[h, m, 0]
        r1 = rand_ref[h, m, 1]
        r2 = rand_ref[h, m, 2]
        qm = q_ref[0, 0, pl.ds((m + 1) * _BLK, _BLK)]
        qm = (qm * _SCALE).astype(jnp.bfloat16)
        mx = row_bound(qm)
        s_g = _dot_t(qm, kg_ref[...])
        s_w = _dot_t(qm, kb_ref[pl.ds(m * _BLK, 3 * _BLK)])
        s_0 = _dot_t(qm, kb_ref[pl.ds(r0 * _BLK, _BLK)])
        s_1 = _dot_t(qm, kb_ref[pl.ds(r1 * _BLK, _BLK)])
        s_2 = _dot_t(qm, kb_ref[pl.ds(r2 * _BLK, _BLK)])
        es = [jnp.exp(sp - mx).astype(jnp.bfloat16)
              for sp in (s_g, s_w, s_0, s_1, s_2)]
        return (es, r0, r1, r2)

    def pv_phase(m, state):
        es, r0, r1, r2 = state
        acc = _dot(es[0], vg_ref[...])
        acc = acc + _dot(es[1], vp_ref[pl.ds(m * _BLK, 3 * _BLK)])
        acc = acc + _dot(es[2], vp_ref[pl.ds(r0 * _BLK, _BLK)])
        acc = acc + _dot(es[3], vp_ref[pl.ds(r1 * _BLK, _BLK)])
        acc = acc + _dot(es[4], vp_ref[pl.ds(r2 * _BLK, _BLK)])
        o_ref[0, 0, pl.ds((m + 1) * _BLK, _BLK)] = (
            acc[:, 0:_DH] / acc[:, _DH:_DH + 1]
        )

    def group(ms):
        states = [qk_phase(m) for m in ms]
        for m, st in zip(ms, states):
            pv_phase(m, st)

    def body(i, carry):
        group([4 * i + j for j in range(4)])
        return carry

    jax.lax.fori_loop(0, 15, body, 0)
    group([60, 61])


def kernel(q, k, v, rand_attn):
    rand = rand_attn.astype(jnp.int32)

    def _spec(b, h, rand_ref):
        return (b, h, 0, 0)

    qkv_spec = pl.BlockSpec((1, 1, _S, _DH), _spec)
    out = pl.pallas_call(
        _bigbird_kernel,
        grid_spec=pltpu.PrefetchScalarGridSpec(
            num_scalar_prefetch=1,
            grid=(_B, _H),
            in_specs=[qkv_spec, qkv_spec, qkv_spec],
            out_specs=qkv_spec,
            scratch_shapes=[
                pltpu.VMEM((_S, _DH), jnp.bfloat16),
                pltpu.VMEM((_S, _VP), jnp.bfloat16),
                pltpu.VMEM((2 * _BLK, _DH), jnp.bfloat16),
                pltpu.VMEM((2 * _BLK, _VP), jnp.bfloat16),
            ],
        ),
        out_shape=jax.ShapeDtypeStruct((_B, _H, _S, _DH), jnp.float32),
        compiler_params=pltpu.CompilerParams(
            dimension_semantics=("parallel", "parallel"),
        ),
    )(rand, q, k, v)
    return out


# speedup vs baseline: 1.8785x; 1.0585x over previous
import numpy as np
import jax
import jax.numpy as jnp
from jax.experimental import pallas as pl
from jax.experimental.pallas import tpu as pltpu

_B, _H, _S, _DH, _BLK = 2, 16, 4096, 64, 64
_NB = _S // _BLK
_R = 3
_M = _NB - 2
_VP = 128
_SCALE = 1.0 / np.sqrt(_DH)
_BOUND_PAD = 1.01


def _dot_t(a, b):
    return jax.lax.dot_general(
        a, b, (((1,), (1,)), ((), ())), preferred_element_type=jnp.float32
    )


def _dot(a, b):
    return jnp.dot(a, b, preferred_element_type=jnp.float32)


def _bigbird_kernel(rand_ref, q_ref, k_ref, v_ref, o_ref,
                    kb_ref, vp_ref, kg_ref, vg_ref):
    h = pl.program_id(1)

    kb_ref[...] = k_ref[0, 0].astype(jnp.bfloat16)
    vp_ref[:, 0:_DH] = v_ref[0, 0].astype(jnp.bfloat16)
    vp_ref[:, _DH:_VP] = jnp.zeros((_S, _VP - _DH), jnp.bfloat16)
    vp_ref[:, _DH:_DH + 1] = jnp.ones((_S, 1), jnp.bfloat16)
    kg_ref[0:_BLK] = kb_ref[0:_BLK]
    kg_ref[_BLK:2 * _BLK] = kb_ref[_S - _BLK:_S]
    vg_ref[0:_BLK] = vp_ref[0:_BLK]
    vg_ref[_BLK:2 * _BLK] = vp_ref[_S - _BLK:_S]

    kf = k_ref[0, 0]
    kmax = jnp.sqrt(jnp.max(jnp.sum(kf * kf, axis=-1))) * _BOUND_PAD

    def row_bound(q_rows):
        qf = q_rows.astype(jnp.float32)
        return jnp.sqrt(
            jnp.sum(qf * qf, axis=-1, keepdims=True)
        ) * (kmax * _BOUND_PAD)

    qg = jnp.concatenate(
        [q_ref[0, 0, 0:_BLK], q_ref[0, 0, _S - _BLK:_S]], axis=0
    )
    qg = (qg * _SCALE).astype(jnp.bfloat16)
    mx_g = row_bound(qg)
    s = _dot_t(qg, kb_ref[...])
    p = jnp.exp(s - mx_g).astype(jnp.bfloat16)
    res = _dot(p, vp_ref[...])
    og = res[:, 0:_DH] / res[:, _DH:_DH + 1]
    o_ref[0, 0, 0:_BLK] = og[0:_BLK]
    o_ref[0, 0, _S - _BLK:_S] = og[_BLK:]

    def qk_phase(m):
        r0 = rand_ref[h, m, 0]
        r1 = rand_ref[h, m, 1]
        r2 = rand_ref[h, m, 2]
        qm = q_ref[0, 0, pl.ds((m + 1) * _BLK, _BLK)]
        qm = (qm * _SCALE).astype(jnp.bfloat16)
        mx = row_bound(qm)
        s_g = _dot_t(qm, kg_ref[...])
        s_w = _dot_t(qm, kb_ref[pl.ds(m * _BLK, 3 * _BLK)])
        s_0 = _dot_t(qm, kb_ref[pl.ds(r0 * _BLK, _BLK)])
        s_1 = _dot_t(qm, kb_ref[pl.ds(r1 * _BLK, _BLK)])
        s_2 = _dot_t(qm, kb_ref[pl.ds(r2 * _BLK, _BLK)])
        es = [jnp.exp(sp - mx).astype(jnp.bfloat16)
              for sp in (s_g, s_w, s_0, s_1, s_2)]
        return (es, r0, r1, r2)

    def pv_phase(m, state):
        es, r0, r1, r2 = state
        acc = _dot(es[0], vg_ref[...])
        acc = acc + _dot(es[1], vp_ref[pl.ds(m * _BLK, 3 * _BLK)])
        acc = acc + _dot(es[2], vp_ref[pl.ds(r0 * _BLK, _BLK)])
        acc = acc + _dot(es[3], vp_ref[pl.ds(r1 * _BLK, _BLK)])
        acc = acc + _dot(es[4], vp_ref[pl.ds(r2 * _BLK, _BLK)])
        o_ref[0, 0, pl.ds((m + 1) * _BLK, _BLK)] = (
            acc[:, 0:_DH] / acc[:, _DH:_DH + 1]
        )

    def group(ms):
        states = [qk_phase(m) for m in ms]
        for m, st in zip(ms, states):
            pv_phase(m, st)

    def body(i, carry):
        group([6 * i + j for j in range(6)])
        return carry

    jax.lax.fori_loop(0, 10, body, 0)
    group([60, 61])


def kernel(q, k, v, rand_attn):
    rand = rand_attn.astype(jnp.int32)

    def _spec(b, h, rand_ref):
        return (b, h, 0, 0)

    qkv_spec = pl.BlockSpec((1, 1, _S, _DH), _spec)
    out = pl.pallas_call(
        _bigbird_kernel,
        grid_spec=pltpu.PrefetchScalarGridSpec(
            num_scalar_prefetch=1,
            grid=(_B, _H),
            in_specs=[qkv_spec, qkv_spec, qkv_spec],
            out_specs=qkv_spec,
            scratch_shapes=[
                pltpu.VMEM((_S, _DH), jnp.bfloat16),
                pltpu.VMEM((_S, _VP), jnp.bfloat16),
                pltpu.VMEM((2 * _BLK, _DH), jnp.bfloat16),
                pltpu.VMEM((2 * _BLK, _VP), jnp.bfloat16),
            ],
        ),
        out_shape=jax.ShapeDtypeStruct((_B, _H, _S, _DH), jnp.float32),
        compiler_params=pltpu.CompilerParams(
            dimension_semantics=("parallel", "parallel"),
        ),
    )(rand, q, k, v)
    return out
